# baseline (speedup 1.0000x reference)
import jax
import jax.numpy as jnp
from jax.experimental import pallas as pl
from jax.experimental.pallas import tpu as pltpu

_MAX_SEQ_LENGTH = 2048
_CHUNK_S = 512


def _pad_dma_body(x_hbm, o_hbm, bufs, zeros_vmem, in_sem, out_sem, zero_sem):
    b, s, f = x_hbm.shape
    pad = _MAX_SEQ_LENGTH - s
    cs = _CHUNK_S
    cpr = s // cs
    n = b * cpr
    zs = zeros_vmem.shape[1]
    zpr = pad // zs
    n_zero = b * zpr

    def in_copy(t):
        i, j = divmod(t, cpr)
        return pltpu.make_async_copy(
            x_hbm.at[pl.ds(i, 1), pl.ds(j * cs, cs)], bufs.at[t], in_sem
        )

    def out_copy(t):
        i, j = divmod(t, cpr)
        return pltpu.make_async_copy(
            bufs.at[t], o_hbm.at[pl.ds(i, 1), pl.ds(j * cs, cs)], out_sem
        )

    def zero_copy(k):
        i, j = divmod(k, zpr)
        return pltpu.make_async_copy(
            zeros_vmem, o_hbm.at[pl.ds(i, 1), pl.ds(s + j * zs, zs)], zero_sem
        )

    in_copy(0).start()
    in_copy(1).start()

    zeros_vmem[...] = jnp.zeros_like(zeros_vmem)
    zero_copy(0).start()
    zero_copy(1).start()

    for t in range(2, n):
        in_copy(t).start()

    for t in range(n):
        in_copy(t).wait()
        out_copy(t).start()
        k = t + 2
        if k < n_zero:
            zero_copy(k).start()

    for t in range(n):
        out_copy(t).wait()
    for k in range(n_zero):
        zero_copy(k).wait()


def kernel(x):
    b, s, f = x.shape
    out_s = _MAX_SEQ_LENGTH
    pad = out_s - s
    cs = _CHUNK_S
    n = (s // cs) * b

    return pl.pallas_call(
        _pad_dma_body,
        in_specs=[pl.BlockSpec(memory_space=pltpu.MemorySpace.HBM)],
        out_specs=pl.BlockSpec(memory_space=pltpu.MemorySpace.HBM),
        out_shape=jax.ShapeDtypeStruct((b, out_s, f), x.dtype),
        scratch_shapes=[
            pltpu.VMEM((n, 1, cs, f), x.dtype),
            pltpu.VMEM((1, _CHUNK_S, f), x.dtype),
            pltpu.SemaphoreType.DMA,
            pltpu.SemaphoreType.DMA,
            pltpu.SemaphoreType.DMA,
        ],
    )(x)
